# initial kernel scaffold (unmeasured)
import jax
import jax.numpy as jnp
from jax import lax
from jax.experimental import pallas as pl
from jax.experimental.pallas import tpu as pltpu

N_DEV = 8
CHUNK_M = 1024


def kernel(x, w_mat):
    k_loc, n = w_mat.shape
    m_glob = x.shape[0]

    def body(x_ref, w_ref, out_ref, comm_ref, send_sems, recv_sems):
        my_pos = lax.axis_index("i")
        left = lax.rem(my_pos - 1 + N_DEV, N_DEV)
        right = lax.rem(my_pos + 1, N_DEV)

        barrier_sem = pltpu.get_barrier_semaphore()
        for nbr in (left, right):
            pl.semaphore_signal(
                barrier_sem, inc=1,
                device_id=(nbr,), device_id_type=pl.DeviceIdType.MESH,
            )
        pl.semaphore_wait(barrier_sem, 2)

        def contrib(chunk_idx):
            xb = x_ref[pl.ds(chunk_idx * CHUNK_M, CHUNK_M), :]
            return jnp.dot(xb, w_ref[:, :], preferred_element_type=jnp.float32)

        c0 = lax.rem(my_pos - 1 + N_DEV, N_DEV)
        comm_ref[0, :, :] = contrib(c0)

        for s in range(N_DEV - 1):
            send_slot = s % 2
            recv_slot = (s + 1) % 2
            rdma = pltpu.make_async_remote_copy(
                src_ref=comm_ref.at[send_slot],
                dst_ref=comm_ref.at[recv_slot],
                send_sem=send_sems.at[send_slot],
                recv_sem=recv_sems.at[recv_slot],
                device_id=(right,),
                device_id_type=pl.DeviceIdType.MESH,
            )
            rdma.start()
            c = lax.rem(my_pos - 2 - s + 2 * N_DEV, N_DEV)
            mine = contrib(c)
            rdma.wait()
            if s < N_DEV - 2:
                comm_ref[recv_slot, :, :] = comm_ref[recv_slot, :, :] + mine
            else:
                out_ref[:, :] = comm_ref[recv_slot, :, :] + mine

    return pl.pallas_call(
        body,
        out_shape=jax.ShapeDtypeStruct((k_loc, n), jnp.float32),
        in_specs=[
            pl.BlockSpec(memory_space=pltpu.VMEM),
            pl.BlockSpec(memory_space=pltpu.VMEM),
        ],
        out_specs=pl.BlockSpec(memory_space=pltpu.VMEM),
        scratch_shapes=[
            pltpu.VMEM((2, CHUNK_M, n), jnp.float32),
            pltpu.SemaphoreType.DMA((2,)),
            pltpu.SemaphoreType.DMA((2,)),
        ],
        compiler_params=pltpu.CompilerParams(collective_id=0),
    )(x, w_mat)


# baseline (device time: 1397204 ns/iter reference)
import jax
import jax.numpy as jnp
from jax import lax
from jax.experimental import pallas as pl
from jax.experimental.pallas import tpu as pltpu

N_DEV = 8
CHUNK_M = 1024
N_SEG = 2


def kernel(x, w_mat):
    k_loc, n = w_mat.shape
    seg_n = n // N_SEG

    def body(x_ref, w_ref, out_ref, xb_ref, comm_ref,
             xb_sem, send_sems, recv_sems):
        my_pos = lax.axis_index("i")
        left = lax.rem(my_pos - 1 + N_DEV, N_DEV)
        right = lax.rem(my_pos + 1, N_DEV)

        barrier_sem = pltpu.get_barrier_semaphore()

        def neighbor_barrier():
            for nbr in (left, right):
                pl.semaphore_signal(
                    barrier_sem, inc=1,
                    device_id=(nbr,), device_id_type=pl.DeviceIdType.MESH,
                )
            pl.semaphore_wait(barrier_sem, 2)

        def load_xb(chunk_idx):
            cp = pltpu.make_async_copy(
                x_ref.at[pl.ds(chunk_idx * CHUNK_M, CHUNK_M), :],
                xb_ref,
                xb_sem,
            )
            cp.start()
            cp.wait()

        for g in range(N_SEG):
            col = pl.ds(g * seg_n, seg_n)

            neighbor_barrier()

            c0 = lax.rem(my_pos - 1 + N_DEV, N_DEV)
            load_xb(c0)
            comm_ref[0, :, :] = jnp.dot(
                xb_ref[:, :], w_ref[:, col],
                preferred_element_type=jnp.float32,
            )

            for s in range(N_DEV - 1):
                send_slot = s % 2
                recv_slot = (s + 1) % 2
                rdma = pltpu.make_async_remote_copy(
                    src_ref=comm_ref.at[send_slot],
                    dst_ref=comm_ref.at[recv_slot],
                    send_sem=send_sems.at[send_slot],
                    recv_sem=recv_sems.at[recv_slot],
                    device_id=(right,),
                    device_id_type=pl.DeviceIdType.MESH,
                )
                rdma.start()
                c = lax.rem(my_pos - 2 - s + 2 * N_DEV, N_DEV)
                load_xb(c)
                rdma.wait()
                mine = jnp.dot(
                    xb_ref[:, :], w_ref[:, col],
                    preferred_element_type=jnp.float32,
                )
                if s < N_DEV - 2:
                    comm_ref[recv_slot, :, :] = comm_ref[recv_slot, :, :] + mine
                else:
                    out_ref[:, col] = comm_ref[recv_slot, :, :] + mine

    return pl.pallas_call(
        body,
        out_shape=jax.ShapeDtypeStruct((k_loc, n), jnp.float32),
        in_specs=[
            pl.BlockSpec(memory_space=pl.ANY),
            pl.BlockSpec(memory_space=pltpu.VMEM),
        ],
        out_specs=pl.BlockSpec(memory_space=pltpu.VMEM),
        scratch_shapes=[
            pltpu.VMEM((CHUNK_M, CHUNK_M), jnp.float32),
            pltpu.VMEM((2, CHUNK_M, n // N_SEG), jnp.float32),
            pltpu.SemaphoreType.DMA,
            pltpu.SemaphoreType.DMA((2,)),
            pltpu.SemaphoreType.DMA((2,)),
        ],
        compiler_params=pltpu.CompilerParams(
            collective_id=0,
            vmem_limit_bytes=64 * 1024 * 1024,
        ),
    )(x, w_mat)


# device time: 751128 ns/iter; 1.8601x vs baseline; 1.8601x over previous
import jax
import jax.numpy as jnp
from jax import lax
from jax.experimental import pallas as pl
from jax.experimental.pallas import tpu as pltpu

N_DEV = 8
CHUNK_M = 1024


def kernel(x, w_mat):
    k_loc, n = w_mat.shape
    seg_n = n // 2

    def body(x_ref, w_ref, out_ref, xb_r, xb_l, comm_r, comm_l,
             xbr_sem, xbl_sem, out_sems,
             send_r, recv_r, send_l, recv_l):
        my_pos = lax.axis_index("i")
        left = lax.rem(my_pos - 1 + N_DEV, N_DEV)
        right = lax.rem(my_pos + 1, N_DEV)

        col_r = pl.ds(0, seg_n)
        col_l = pl.ds(seg_n, seg_n)

        barrier_sem = pltpu.get_barrier_semaphore()
        for nbr in (left, right):
            pl.semaphore_signal(
                barrier_sem, inc=1,
                device_id=(nbr,), device_id_type=pl.DeviceIdType.MESH,
            )
        pl.semaphore_wait(barrier_sem, 2)

        def stage(chunk_idx, buf, sem):
            cp = pltpu.make_async_copy(
                x_ref.at[pl.ds(chunk_idx * CHUNK_M, CHUNK_M), :], buf, sem,
            )
            cp.start()
            return cp

        c0_r = lax.rem(my_pos - 1 + N_DEV, N_DEV)
        c0_l = lax.rem(my_pos + 1, N_DEV)
        cp_r = stage(c0_r, xb_r, xbr_sem)
        cp_l = stage(c0_l, xb_l, xbl_sem)
        cp_r.wait()
        comm_r[0, :, :] = jnp.dot(
            xb_r[:, :], w_ref[:, col_r], preferred_element_type=jnp.float32)
        cp_l.wait()
        comm_l[0, :, :] = jnp.dot(
            xb_l[:, :], w_ref[:, col_l], preferred_element_type=jnp.float32)

        for s in range(N_DEV - 1):
            send_slot = s % 2
            recv_slot = (s + 1) % 2
            rdma_r = pltpu.make_async_remote_copy(
                src_ref=comm_r.at[send_slot],
                dst_ref=comm_r.at[recv_slot],
                send_sem=send_r.at[send_slot],
                recv_sem=recv_r.at[recv_slot],
                device_id=(right,),
                device_id_type=pl.DeviceIdType.MESH,
            )
            rdma_l = pltpu.make_async_remote_copy(
                src_ref=comm_l.at[send_slot],
                dst_ref=comm_l.at[recv_slot],
                send_sem=send_l.at[send_slot],
                recv_sem=recv_l.at[recv_slot],
                device_id=(left,),
                device_id_type=pl.DeviceIdType.MESH,
            )
            rdma_r.start()
            rdma_l.start()

            c_r = lax.rem(my_pos - 2 - s + 2 * N_DEV, N_DEV)
            c_l = lax.rem(my_pos + 2 + s, N_DEV)
            cp_r = stage(c_r, xb_r, xbr_sem)
            cp_l = stage(c_l, xb_l, xbl_sem)

            cp_r.wait()
            rdma_r.wait()
            comm_r[recv_slot, :, :] = comm_r[recv_slot, :, :] + jnp.dot(
                xb_r[:, :], w_ref[:, col_r],
                preferred_element_type=jnp.float32)
            cp_l.wait()
            rdma_l.wait()
            comm_l[recv_slot, :, :] = comm_l[recv_slot, :, :] + jnp.dot(
                xb_l[:, :], w_ref[:, col_l],
                preferred_element_type=jnp.float32)

            if s == N_DEV - 2:
                wr = pltpu.make_async_copy(
                    comm_r.at[recv_slot], out_ref.at[:, col_r],
                    out_sems.at[0])
                wl = pltpu.make_async_copy(
                    comm_l.at[recv_slot], out_ref.at[:, col_l],
                    out_sems.at[1])
                wr.start()
                wl.start()
                wr.wait()
                wl.wait()

    return pl.pallas_call(
        body,
        out_shape=jax.ShapeDtypeStruct((k_loc, n), jnp.float32),
        in_specs=[
            pl.BlockSpec(memory_space=pl.ANY),
            pl.BlockSpec(memory_space=pltpu.VMEM),
        ],
        out_specs=pl.BlockSpec(memory_space=pl.ANY),
        scratch_shapes=[
            pltpu.VMEM((CHUNK_M, CHUNK_M), jnp.float32),
            pltpu.VMEM((CHUNK_M, CHUNK_M), jnp.float32),
            pltpu.VMEM((2, CHUNK_M, n // 2), jnp.float32),
            pltpu.VMEM((2, CHUNK_M, n // 2), jnp.float32),
            pltpu.SemaphoreType.DMA,
            pltpu.SemaphoreType.DMA,
            pltpu.SemaphoreType.DMA((2,)),
            pltpu.SemaphoreType.DMA((2,)),
            pltpu.SemaphoreType.DMA((2,)),
            pltpu.SemaphoreType.DMA((2,)),
            pltpu.SemaphoreType.DMA((2,)),
        ],
        compiler_params=pltpu.CompilerParams(
            collective_id=0,
            vmem_limit_bytes=64 * 1024 * 1024,
        ),
    )(x, w_mat)


# device time: 670794 ns/iter; 2.0829x vs baseline; 1.1198x over previous
import jax
import jax.numpy as jnp
from jax import lax
from jax.experimental import pallas as pl
from jax.experimental.pallas import tpu as pltpu

N_DEV = 8
CHUNK_M = 1024
N_RING = 4


def kernel(x, w_mat):
    k_loc, n = w_mat.shape
    qn = n // N_RING

    def body(x_ref, w_ref, out_ref, xb_r, xb_l, comm,
             xbr_sem, xbl_sem, out_sems, send_sems, recv_sems):
        my_pos = lax.axis_index("i")
        left = lax.rem(my_pos - 1 + N_DEV, N_DEV)
        right = lax.rem(my_pos + 1, N_DEV)

        barrier_sem = pltpu.get_barrier_semaphore()
        for nbr in (left, right):
            pl.semaphore_signal(
                barrier_sem, inc=1,
                device_id=(nbr,), device_id_type=pl.DeviceIdType.MESH,
            )
        pl.semaphore_wait(barrier_sem, 2)

        def stage(chunk_idx, buf, sem):
            cp = pltpu.make_async_copy(
                x_ref.at[pl.ds(chunk_idx * CHUNK_M, CHUNK_M), :], buf, sem,
            )
            cp.start()
            return cp

        def mk(r, s):
            return pltpu.make_async_remote_copy(
                src_ref=comm.at[r, s % 2],
                dst_ref=comm.at[r, (s + 1) % 2],
                send_sem=send_sems.at[r, s % 2],
                recv_sem=recv_sems.at[r, (s + 1) % 2],
                device_id=(right,) if r < 2 else (left,),
                device_id_type=pl.DeviceIdType.MESH,
            )

        def accum(r, slot, xb):
            col = pl.ds(r * qn, qn)
            comm[r, slot, :, :] = comm[r, slot, :, :] + jnp.dot(
                xb[:, :], w_ref[:, col], preferred_element_type=jnp.float32)

        def chunk_r(s):
            return lax.rem(my_pos - 2 - s + 2 * N_DEV, N_DEV)

        def chunk_l(s):
            return lax.rem(my_pos + 2 + s, N_DEV)

        cp_r = stage(lax.rem(my_pos - 1 + N_DEV, N_DEV), xb_r, xbr_sem)
        cp_l = stage(lax.rem(my_pos + 1, N_DEV), xb_l, xbl_sem)
        cp_r.wait()
        for r in (0, 1):
            col = pl.ds(r * qn, qn)
            comm[r, 0, :, :] = jnp.dot(
                xb_r[:, :], w_ref[:, col], preferred_element_type=jnp.float32)
            mk(r, 0).start()
        cp_l.wait()
        for r in (2, 3):
            col = pl.ds(r * qn, qn)
            comm[r, 0, :, :] = jnp.dot(
                xb_l[:, :], w_ref[:, col], preferred_element_type=jnp.float32)
            mk(r, 0).start()

        cp_r = stage(chunk_r(0), xb_r, xbr_sem)
        cp_l = stage(chunk_l(0), xb_l, xbl_sem)

        for s in range(N_DEV - 1):
            recv_slot = (s + 1) % 2
            last = s == N_DEV - 2

            cp_r.wait()
            mk(0, s).wait()
            accum(0, recv_slot, xb_r)
            if not last:
                mk(0, s + 1).start()

            cp_l.wait()
            mk(2, s).wait()
            accum(2, recv_slot, xb_l)
            if not last:
                mk(2, s + 1).start()

            mk(1, s).wait()
            accum(1, recv_slot, xb_r)
            if not last:
                mk(1, s + 1).start()
                cp_r = stage(chunk_r(s + 1), xb_r, xbr_sem)

            mk(3, s).wait()
            accum(3, recv_slot, xb_l)
            if not last:
                mk(3, s + 1).start()
                cp_l = stage(chunk_l(s + 1), xb_l, xbl_sem)

        wbs = []
        for r in range(N_RING):
            wb = pltpu.make_async_copy(
                comm.at[r, (N_DEV - 1) % 2],
                out_ref.at[:, pl.ds(r * qn, qn)],
                out_sems.at[r],
            )
            wb.start()
            wbs.append(wb)
        for wb in wbs:
            wb.wait()

    return pl.pallas_call(
        body,
        out_shape=jax.ShapeDtypeStruct((k_loc, n), jnp.float32),
        in_specs=[
            pl.BlockSpec(memory_space=pl.ANY),
            pl.BlockSpec(memory_space=pltpu.VMEM),
        ],
        out_specs=pl.BlockSpec(memory_space=pl.ANY),
        scratch_shapes=[
            pltpu.VMEM((CHUNK_M, CHUNK_M), jnp.float32),
            pltpu.VMEM((CHUNK_M, CHUNK_M), jnp.float32),
            pltpu.VMEM((N_RING, 2, CHUNK_M, n // N_RING), jnp.float32),
            pltpu.SemaphoreType.DMA,
            pltpu.SemaphoreType.DMA,
            pltpu.SemaphoreType.DMA((N_RING,)),
            pltpu.SemaphoreType.DMA((N_RING, 2)),
            pltpu.SemaphoreType.DMA((N_RING, 2)),
        ],
        compiler_params=pltpu.CompilerParams(
            collective_id=0,
            vmem_limit_bytes=64 * 1024 * 1024,
        ),
    )(x, w_mat)
